# baseline (speedup 1.0000x reference)
import functools

import jax
import jax.numpy as jnp
from jax import lax
from jax.experimental import pallas as pl
from jax.experimental.pallas import tpu as pltpu
from jax.experimental.pallas import tpu_sc as plsc

N = 10000
K = 16
C = 128
NK = N * K
EPS = 1e-5
_INV_NK = 1.0 / NK


def _sc_gather(table, idx, chunk=40):
  row = table.shape[1:]
  info = plsc.get_sparse_core_info()
  nc = info.num_cores
  nw = nc * info.num_subcores
  per_w = NK // nw
  n_chunks = per_w // chunk
  mesh = plsc.VectorSubcoreMesh(core_axis_name="c", subcore_axis_name="s")

  @functools.partial(
      pl.kernel,
      mesh=mesh,
      out_type=jax.ShapeDtypeStruct((NK,) + row, table.dtype),
      scratch_types=[
          pltpu.VMEM((chunk,), jnp.int32),
          pltpu.VMEM((chunk,) + row, table.dtype),
          pltpu.SemaphoreType.DMA,
      ],
  )
  def gk(table_hbm, idx_hbm, out_hbm, idx_v, rows_v, sem):
    wid = lax.axis_index("s") * nc + lax.axis_index("c")
    w_base = wid * per_w

    def body(t, carry):
      base = w_base + t * chunk
      pltpu.sync_copy(idx_hbm.at[pl.ds(base, chunk)], idx_v)
      pltpu.async_copy(table_hbm.at[idx_v], rows_v, sem).wait()
      pltpu.sync_copy(rows_v, out_hbm.at[pl.ds(base, chunk)])
      return carry

    lax.fori_loop(0, n_chunks, body, 0)

  return gk(table, idx)


def _moments(cg):
  nb = 20
  rb = NK // nb

  def body(cg_ref, m1_ref, m2_ref):
    @pl.when(pl.program_id(0) == 0)
    def _init():
      m1_ref[...] = jnp.zeros_like(m1_ref)
      m2_ref[...] = jnp.zeros_like(m2_ref)

    blk = cg_ref[...][:, :8]
    m1_ref[...] += jnp.sum(blk, axis=0, keepdims=True)
    m2_ref[...] += lax.dot_general(blk, blk, (((0,), (0,)), ((), ())),
                                   preferred_element_type=jnp.float32)

  return pl.pallas_call(
      body,
      grid=(nb,),
      in_specs=[pl.BlockSpec((rb, C), lambda i: (i, 0))],
      out_specs=[pl.BlockSpec((1, 8), lambda i: (0, 0)),
                 pl.BlockSpec((8, 8), lambda i: (0, 0))],
      out_shape=[jax.ShapeDtypeStruct((1, 8), jnp.float32),
                 jax.ShapeDtypeStruct((8, 8), jnp.float32)],
  )(cg)


def _tables(spa_t, spe_t, coordp, m1, m2, wqkv, bqkv, wp1, bp1, gp, bpn,
            wp2, bp2):
  nb = 10
  rb = N // nb

  def body(spa_ref, spe_ref, cp_ref, m1_ref, m2_ref, wqkv_ref, bqkv_ref,
           wp1_ref, bp1_ref, gp_ref, bpn_ref, wp2_ref, bp2_ref,
           t_ref, xq_ref):
    spa_b = spa_ref[...]
    spe_b = spe_ref[...]
    cp_b = cp_ref[...]
    mean_x = m1_ref[...] * _INV_NK
    exx = m2_ref[...] * _INV_NK
    eye8 = jnp.eye(8, dtype=jnp.float32)

    def proj(x, i):
      return lax.dot_general(x, wqkv_ref[i], (((1,), (1,)), ((), ())),
                             preferred_element_type=jnp.float32
                             ) + bqkv_ref[i].reshape(1, C)

    def pbr(l):
      w1 = wp1_ref[l]
      b1 = bp1_ref[l].reshape(1, 8)
      g1 = gp_ref[l].reshape(1, 8)
      bn1 = bpn_ref[l].reshape(1, 8)
      c1m = lax.dot_general(mean_x, w1, (((1,), (1,)), ((), ()))) + b1
      t1 = lax.dot_general(w1, exx, (((1,), (0,)), ((), ())))
      t2 = lax.dot_general(t1, w1, (((1,), (1,)), ((), ())))
      diag = jnp.sum(t2 * eye8, axis=1).reshape(1, 8)
      ec1sq = diag + 2.0 * b1 * c1m - b1 * b1
      inv = lax.rsqrt(ec1sq - c1m * c1m + EPS)
      c1 = lax.dot_general(cp_b, w1, (((1,), (1,)), ((), ()))) + b1
      c1n = jnp.maximum((c1 - c1m) * inv * g1 + bn1, 0.0)
      return lax.dot_general(c1n, wp2_ref[l], (((1,), (1,)), ((), ())),
                             preferred_element_type=jnp.float32
                             ) + bp2_ref[l].reshape(1, C)

    q_a = proj(spa_b, 0)
    k_a = proj(spe_b, 1)
    v_a = proj(spa_b, 2)
    q_e = proj(spe_b, 3)
    k_e = proj(spa_b, 4)
    v_e = proj(spe_b, 5)
    p_a = pbr(0)
    p_e = pbr(1)
    t_ref[...] = jnp.concatenate(
        [k_a + p_a, k_e + p_e, v_a + p_a, v_e + p_e],
        axis=1).astype(jnp.bfloat16)
    xq_ref[...] = jnp.concatenate([q_a, q_e], axis=1)

  z = lambda i: (i, 0)
  z3 = lambda i: (0, 0, 0)
  c0 = lambda i: (0, 0)
  return pl.pallas_call(
      body,
      grid=(nb,),
      in_specs=[
          pl.BlockSpec((rb, C), z), pl.BlockSpec((rb, C), z),
          pl.BlockSpec((rb, 8), z),
          pl.BlockSpec((1, 8), c0), pl.BlockSpec((8, 8), c0),
          pl.BlockSpec((6, C, C), z3), pl.BlockSpec((6, C), c0),
          pl.BlockSpec((2, 8, 8), z3), pl.BlockSpec((2, 8), c0),
          pl.BlockSpec((2, 8), c0), pl.BlockSpec((2, 8), c0),
          pl.BlockSpec((2, C, 8), z3), pl.BlockSpec((2, C), c0),
      ],
      out_specs=[pl.BlockSpec((rb, 512), z), pl.BlockSpec((rb, 256), z)],
      out_shape=[jax.ShapeDtypeStruct((N, 512), jnp.bfloat16),
                 jax.ShapeDtypeStruct((N, 256), jnp.float32)],
  )(spa_t, spe_t, coordp, m1, m2, wqkv, bqkv, wp1, bp1, gp, bpn, wp2, bp2)


def _stats(g, xq):
  nb = 50
  rb = N // nb

  def body(g_ref, xq_ref, s1_ref, s2_ref):
    @pl.when(pl.program_id(0) == 0)
    def _init():
      s1_ref[...] = jnp.zeros_like(s1_ref)
      s2_ref[...] = jnp.zeros_like(s2_ref)

    gb = g_ref[...].astype(jnp.float32).reshape(rb, K, 256)
    w = gb - xq_ref[...][:, None, :]
    s1_ref[...] += jnp.sum(w, axis=(0, 1)).reshape(1, 256)
    s2_ref[...] += jnp.sum(w * w, axis=(0, 1)).reshape(1, 256)

  return pl.pallas_call(
      body,
      grid=(nb,),
      in_specs=[pl.BlockSpec((rb * K, 256), lambda i: (i, 0)),
                pl.BlockSpec((rb, 256), lambda i: (i, 0))],
      out_specs=[pl.BlockSpec((1, 256), lambda i: (0, 0)),
                 pl.BlockSpec((1, 256), lambda i: (0, 0))],
      out_shape=[jax.ShapeDtypeStruct((1, 256), jnp.float32),
                 jax.ShapeDtypeStruct((1, 256), jnp.float32)],
  )(g, xq)


def _ypass(g, xq, s1, s2, g1, b1, w1, bb1):
  nb = 50
  rb = N // nb

  def body(g_ref, xq_ref, s1_ref, s2_ref, g1_ref, b1_ref, w1_ref, bb1_ref,
           y_ref, s3_ref, s4_ref):
    @pl.when(pl.program_id(0) == 0)
    def _init():
      s3_ref[...] = jnp.zeros_like(s3_ref)
      s4_ref[...] = jnp.zeros_like(s4_ref)

    mean = s1_ref[...] * _INV_NK
    var = s2_ref[...] * _INV_NK - mean * mean
    scale = lax.rsqrt(var + EPS) * g1_ref[...]
    gb = g_ref[...].astype(jnp.float32).reshape(rb, K, 256)
    w = (gb - xq_ref[...][:, None, :]).reshape(rb * K, 256)
    r = jnp.maximum((w - mean) * scale + b1_ref[...], 0.0)
    y = lax.dot_general(r, w1_ref[...], (((1,), (0,)), ((), ())),
                        preferred_element_type=jnp.float32) + bb1_ref[...]
    y_ref[...] = y
    s3_ref[...] += jnp.sum(y, axis=0, keepdims=True)
    s4_ref[...] += jnp.sum(y * y, axis=0, keepdims=True)

  z = lambda i: (i, 0)
  c0 = lambda i: (0, 0)
  return pl.pallas_call(
      body,
      grid=(nb,),
      in_specs=[
          pl.BlockSpec((rb * K, 256), z), pl.BlockSpec((rb, 256), z),
          pl.BlockSpec((1, 256), c0), pl.BlockSpec((1, 256), c0),
          pl.BlockSpec((1, 256), c0), pl.BlockSpec((1, 256), c0),
          pl.BlockSpec((256, 16), c0), pl.BlockSpec((1, 16), c0),
      ],
      out_specs=[pl.BlockSpec((rb * K, 16), z),
                 pl.BlockSpec((1, 16), c0), pl.BlockSpec((1, 16), c0)],
      out_shape=[jax.ShapeDtypeStruct((NK, 16), jnp.float32),
                 jax.ShapeDtypeStruct((1, 16), jnp.float32),
                 jax.ShapeDtypeStruct((1, 16), jnp.float32)],
  )(g, xq, s1, s2, g1, b1, w1, bb1)


def _final(y, g, sp, s3, s4, g2, b2, w2, bb2):
  nb = 50
  rb = N // nb

  def body(y_ref, g_ref, sp_ref, s3_ref, s4_ref, g2_ref, b2_ref, w2_ref,
           bb2_ref, out_ref):
    mean = s3_ref[...] * _INV_NK
    var = s4_ref[...] * _INV_NK - mean * mean
    scale = lax.rsqrt(var + EPS) * g2_ref[...]
    y3 = jnp.maximum((y_ref[...] - mean) * scale + b2_ref[...], 0.0)
    zz = lax.dot_general(y3, w2_ref[...], (((1,), (0,)), ((), ())),
                         preferred_element_type=jnp.float32) + bb2_ref[...]
    z3 = zz.reshape(rb, K, 16)
    zmax = jnp.max(z3, axis=1, keepdims=True)
    e = jnp.exp(z3 - zmax)
    soft = e / jnp.sum(e, axis=1, keepdims=True)
    sa = jnp.concatenate([soft[:, :, :8]] * 16, axis=2)
    se = jnp.concatenate([soft[:, :, 8:]] * 16, axis=2)
    gv = g_ref[...].astype(jnp.float32).reshape(rb, K, 256)
    oa = jnp.sum(gv[:, :, :C] * sa, axis=1)
    oe = jnp.sum(gv[:, :, C:] * se, axis=1)
    out_ref[...] = jnp.concatenate([oa, oe], axis=1) + sp_ref[...]

  z = lambda i: (i, 0)
  c0 = lambda i: (0, 0)
  return pl.pallas_call(
      body,
      grid=(nb,),
      in_specs=[
          pl.BlockSpec((rb * K, 16), z),
          pl.BlockSpec((rb * K, 256), lambda i: (i, 1)),
          pl.BlockSpec((rb, 256), z),
          pl.BlockSpec((1, 16), c0), pl.BlockSpec((1, 16), c0),
          pl.BlockSpec((1, 16), c0), pl.BlockSpec((1, 16), c0),
          pl.BlockSpec((16, 16), c0), pl.BlockSpec((1, 16), c0),
      ],
      out_specs=pl.BlockSpec((rb, 256), z),
      out_shape=jax.ShapeDtypeStruct((N, 256), jnp.float32),
  )(y, g, sp, s3, s4, g2, b2, w2, bb2)


def kernel(coord, spa, spe, neighbor_indices, params):
  pa, pe = params['spa'], params['spe']
  spa_t = spa[0].T
  spe_t = spe[0].T
  coordp = jnp.pad(coord[0], ((0, 0), (0, 5)))
  coordw = jnp.pad(coord[0], ((0, 0), (0, C - 3)))
  idx = neighbor_indices[0].reshape(NK).astype(jnp.int32)

  cg = _sc_gather(coordw, idx)
  m1, m2 = _moments(cg)

  wqkv = jnp.stack([pa['wq'], pa['wk'], pa['wv'],
                    pe['wq'], pe['wk'], pe['wv']])
  bqkv = jnp.stack([pa['bq'], pa['bk'], pa['bv'],
                    pe['bq'], pe['bk'], pe['bv']])
  pad88 = lambda w: jnp.zeros((8, 8), jnp.float32).at[:3, :3].set(w)
  pad8 = lambda v: jnp.zeros((8,), jnp.float32).at[:3].set(v)
  wp1 = jnp.stack([pad88(pa['wp1']), pad88(pe['wp1'])])
  bp1 = jnp.stack([pad8(pa['bp1']), pad8(pe['bp1'])])
  gp = jnp.stack([pad8(pa['gp']), pad8(pe['gp'])])
  bpn = jnp.stack([pad8(pa['bpn']), pad8(pe['bpn'])])
  padw2 = lambda w: jnp.zeros((C, 8), jnp.float32).at[:, :3].set(w)
  wp2 = jnp.stack([padw2(pa['wp2']), padw2(pe['wp2'])])
  bp2 = jnp.stack([pa['bp2'], pe['bp2']])

  t, xq = _tables(spa_t, spe_t, coordp, m1, m2, wqkv, bqkv,
                  wp1, bp1, gp, bpn, wp2, bp2)
  t_i32 = lax.bitcast_convert_type(t.reshape(N, 256, 2), jnp.int32)
  g_i32 = _sc_gather(t_i32, idx)
  g = lax.bitcast_convert_type(g_i32, jnp.bfloat16).reshape(NK, 512)
  s1, s2 = _stats(g, xq)

  g1 = jnp.concatenate([pa['gw1'], pe['gw1']]).reshape(1, 256)
  b1 = jnp.concatenate([pa['bw1'], pe['bw1']]).reshape(1, 256)
  w1 = (jnp.zeros((256, 16), jnp.float32)
        .at[:C, :8].set(pa['ww1'].T).at[C:, 8:].set(pe['ww1'].T))
  bb1 = jnp.concatenate([pa['bww1'], pe['bww1']]).reshape(1, 16)
  y, s3, s4 = _ypass(g, xq, s1, s2, g1, b1, w1, bb1)

  g2 = jnp.concatenate([pa['gw2'], pe['gw2']]).reshape(1, 16)
  b2 = jnp.concatenate([pa['bw2'], pe['bw2']]).reshape(1, 16)
  w2 = (jnp.zeros((16, 16), jnp.float32)
        .at[:8, :8].set(pa['ww2'].T).at[8:, 8:].set(pe['ww2'].T))
  bb2 = jnp.concatenate([pa['bww2'], pe['bww2']]).reshape(1, 16)
  sp = jnp.concatenate([spa_t, spe_t], axis=1)
  out = _final(y, g, sp, s3, s4, g2, b2, w2, bb2)
  return out.T[None]


# speedup vs baseline: 2.1909x; 2.1909x over previous
import functools

import jax
import jax.numpy as jnp
from jax import lax
from jax.experimental import pallas as pl
from jax.experimental.pallas import tpu as pltpu
from jax.experimental.pallas import tpu_sc as plsc

N = 10000
K = 16
C = 128
NK = N * K
EPS = 1e-5
_INV_NK = 1.0 / NK


def _sc_gather(table, idx, chunk=40):
  row = table.shape[1:]
  info = plsc.get_sparse_core_info()
  nc = info.num_cores
  nw = nc * info.num_subcores
  per_w = NK // nw
  n_chunks = per_w // chunk
  mesh = plsc.VectorSubcoreMesh(core_axis_name="c", subcore_axis_name="s")

  @functools.partial(
      pl.kernel,
      mesh=mesh,
      out_type=jax.ShapeDtypeStruct((NK,) + row, table.dtype),
      scratch_types=[
          pltpu.VMEM((chunk,), jnp.int32),
          pltpu.VMEM((chunk,) + row, table.dtype),
          pltpu.SemaphoreType.DMA,
      ],
  )
  def gk(table_hbm, idx_hbm, out_hbm, idx_v, rows_v, sem):
    wid = lax.axis_index("s") * nc + lax.axis_index("c")
    w_base = wid * per_w

    def body(t, carry):
      base = w_base + t * chunk
      pltpu.sync_copy(idx_hbm.at[pl.ds(base, chunk)], idx_v)
      pltpu.async_copy(table_hbm.at[idx_v], rows_v, sem).wait()
      pltpu.sync_copy(rows_v, out_hbm.at[pl.ds(base, chunk)])
      return carry

    lax.fori_loop(0, n_chunks, body, 0)

  return gk(table, idx)


def _moments(cg):
  nb = 20
  rb = NK // nb

  def body(cg_ref, m1_ref, m2_ref):
    @pl.when(pl.program_id(0) == 0)
    def _init():
      m1_ref[...] = jnp.zeros_like(m1_ref)
      m2_ref[...] = jnp.zeros_like(m2_ref)

    blk = cg_ref[...][:, :8]
    m1_ref[...] += jnp.sum(blk, axis=0, keepdims=True)
    m2_ref[...] += lax.dot_general(blk, blk, (((0,), (0,)), ((), ())),
                                   preferred_element_type=jnp.float32)

  return pl.pallas_call(
      body,
      grid=(nb,),
      in_specs=[pl.BlockSpec((rb, C), lambda i: (i, 0))],
      out_specs=[pl.BlockSpec((1, 8), lambda i: (0, 0)),
                 pl.BlockSpec((8, 8), lambda i: (0, 0))],
      out_shape=[jax.ShapeDtypeStruct((1, 8), jnp.float32),
                 jax.ShapeDtypeStruct((8, 8), jnp.float32)],
  )(cg)


def _tables(spa_t, spe_t, coordp, m1, m2, wqkv, bqkv, wp1, bp1, gp, bpn,
            wp2, bp2):
  nb = 10
  rb = N // nb

  def body(spa_ref, spe_ref, cp_ref, m1_ref, m2_ref, wqkv_ref, bqkv_ref,
           wp1_ref, bp1_ref, gp_ref, bpn_ref, wp2_ref, bp2_ref,
           t_ref, xq_ref):
    spa_b = spa_ref[...]
    spe_b = spe_ref[...]
    cp_b = cp_ref[...]
    mean_x = m1_ref[...] * _INV_NK
    exx = m2_ref[...] * _INV_NK
    eye8 = jnp.eye(8, dtype=jnp.float32)

    def proj(x, i):
      return lax.dot_general(x, wqkv_ref[i], (((1,), (1,)), ((), ())),
                             preferred_element_type=jnp.float32
                             ) + bqkv_ref[i].reshape(1, C)

    def pbr(l):
      w1 = wp1_ref[l]
      b1 = bp1_ref[l].reshape(1, 8)
      g1 = gp_ref[l].reshape(1, 8)
      bn1 = bpn_ref[l].reshape(1, 8)
      c1m = lax.dot_general(mean_x, w1, (((1,), (1,)), ((), ()))) + b1
      t1 = lax.dot_general(w1, exx, (((1,), (0,)), ((), ())))
      t2 = lax.dot_general(t1, w1, (((1,), (1,)), ((), ())))
      diag = jnp.sum(t2 * eye8, axis=1).reshape(1, 8)
      ec1sq = diag + 2.0 * b1 * c1m - b1 * b1
      inv = lax.rsqrt(ec1sq - c1m * c1m + EPS)
      c1 = lax.dot_general(cp_b, w1, (((1,), (1,)), ((), ()))) + b1
      c1n = jnp.maximum((c1 - c1m) * inv * g1 + bn1, 0.0)
      return lax.dot_general(c1n, wp2_ref[l], (((1,), (1,)), ((), ())),
                             preferred_element_type=jnp.float32
                             ) + bp2_ref[l].reshape(1, C)

    q_a = proj(spa_b, 0)
    k_a = proj(spe_b, 1)
    v_a = proj(spa_b, 2)
    q_e = proj(spe_b, 3)
    k_e = proj(spa_b, 4)
    v_e = proj(spe_b, 5)
    p_a = pbr(0)
    p_e = pbr(1)
    a_part = jnp.concatenate([k_a + p_a, k_e + p_e], axis=1)
    v_part = jnp.concatenate([v_a + p_a, v_e + p_e], axis=1)

    def bfbits(x):
      xi = lax.bitcast_convert_type(x, jnp.int32)
      return (xi + 0x7FFF + ((xi >> 16) & 1)) >> 16

    t_ref[...] = (bfbits(v_part) << 16) | (bfbits(a_part) & 0xFFFF)
    xq_ref[...] = jnp.concatenate([q_a, q_e], axis=1)

  z = lambda i: (i, 0)
  z3 = lambda i: (0, 0, 0)
  c0 = lambda i: (0, 0)
  return pl.pallas_call(
      body,
      grid=(nb,),
      in_specs=[
          pl.BlockSpec((rb, C), z), pl.BlockSpec((rb, C), z),
          pl.BlockSpec((rb, 8), z),
          pl.BlockSpec((1, 8), c0), pl.BlockSpec((8, 8), c0),
          pl.BlockSpec((6, C, C), z3), pl.BlockSpec((6, C), c0),
          pl.BlockSpec((2, 8, 8), z3), pl.BlockSpec((2, 8), c0),
          pl.BlockSpec((2, 8), c0), pl.BlockSpec((2, 8), c0),
          pl.BlockSpec((2, C, 8), z3), pl.BlockSpec((2, C), c0),
      ],
      out_specs=[pl.BlockSpec((rb, 256), z), pl.BlockSpec((rb, 256), z)],
      out_shape=[jax.ShapeDtypeStruct((N, 256), jnp.int32),
                 jax.ShapeDtypeStruct((N, 256), jnp.float32)],
  )(spa_t, spe_t, coordp, m1, m2, wqkv, bqkv, wp1, bp1, gp, bpn, wp2, bp2)


def _stats(g, xq):
  nb = 50
  rb = N // nb

  def body(g_ref, xq_ref, s1_ref, s2_ref):
    @pl.when(pl.program_id(0) == 0)
    def _init():
      s1_ref[...] = jnp.zeros_like(s1_ref)
      s2_ref[...] = jnp.zeros_like(s2_ref)

    gb = lax.bitcast_convert_type(g_ref[...] << 16,
                                  jnp.float32).reshape(rb, K, 256)
    w = gb - xq_ref[...][:, None, :]
    s1_ref[...] += jnp.sum(w, axis=(0, 1)).reshape(1, 256)
    s2_ref[...] += jnp.sum(w * w, axis=(0, 1)).reshape(1, 256)

  return pl.pallas_call(
      body,
      grid=(nb,),
      in_specs=[pl.BlockSpec((rb * K, 256), lambda i: (i, 0)),
                pl.BlockSpec((rb, 256), lambda i: (i, 0))],
      out_specs=[pl.BlockSpec((1, 256), lambda i: (0, 0)),
                 pl.BlockSpec((1, 256), lambda i: (0, 0))],
      out_shape=[jax.ShapeDtypeStruct((1, 256), jnp.float32),
                 jax.ShapeDtypeStruct((1, 256), jnp.float32)],
  )(g, xq)


def _ypass(g, xq, s1, s2, g1, b1, w1, bb1):
  nb = 50
  rb = N // nb

  def body(g_ref, xq_ref, s1_ref, s2_ref, g1_ref, b1_ref, w1_ref, bb1_ref,
           y_ref, s3_ref, s4_ref):
    @pl.when(pl.program_id(0) == 0)
    def _init():
      s3_ref[...] = jnp.zeros_like(s3_ref)
      s4_ref[...] = jnp.zeros_like(s4_ref)

    mean = s1_ref[...] * _INV_NK
    var = s2_ref[...] * _INV_NK - mean * mean
    scale = lax.rsqrt(var + EPS) * g1_ref[...]
    gb = lax.bitcast_convert_type(g_ref[...] << 16,
                                  jnp.float32).reshape(rb, K, 256)
    w = (gb - xq_ref[...][:, None, :]).reshape(rb * K, 256)
    r = jnp.maximum((w - mean) * scale + b1_ref[...], 0.0)
    y = lax.dot_general(r, w1_ref[...], (((1,), (0,)), ((), ())),
                        preferred_element_type=jnp.float32) + bb1_ref[...]
    y_ref[...] = y
    s3_ref[...] += jnp.sum(y, axis=0, keepdims=True)
    s4_ref[...] += jnp.sum(y * y, axis=0, keepdims=True)

  z = lambda i: (i, 0)
  c0 = lambda i: (0, 0)
  return pl.pallas_call(
      body,
      grid=(nb,),
      in_specs=[
          pl.BlockSpec((rb * K, 256), z), pl.BlockSpec((rb, 256), z),
          pl.BlockSpec((1, 256), c0), pl.BlockSpec((1, 256), c0),
          pl.BlockSpec((1, 256), c0), pl.BlockSpec((1, 256), c0),
          pl.BlockSpec((256, 16), c0), pl.BlockSpec((1, 16), c0),
      ],
      out_specs=[pl.BlockSpec((rb * K, 16), z),
                 pl.BlockSpec((1, 16), c0), pl.BlockSpec((1, 16), c0)],
      out_shape=[jax.ShapeDtypeStruct((NK, 16), jnp.float32),
                 jax.ShapeDtypeStruct((1, 16), jnp.float32),
                 jax.ShapeDtypeStruct((1, 16), jnp.float32)],
  )(g, xq, s1, s2, g1, b1, w1, bb1)


def _final(y, g, sp, s3, s4, g2, b2, w2, bb2):
  nb = 50
  rb = N // nb

  def body(y_ref, g_ref, sp_ref, s3_ref, s4_ref, g2_ref, b2_ref, w2_ref,
           bb2_ref, out_ref):
    mean = s3_ref[...] * _INV_NK
    var = s4_ref[...] * _INV_NK - mean * mean
    scale = lax.rsqrt(var + EPS) * g2_ref[...]
    y3 = jnp.maximum((y_ref[...] - mean) * scale + b2_ref[...], 0.0)
    zz = lax.dot_general(y3, w2_ref[...], (((1,), (0,)), ((), ())),
                         preferred_element_type=jnp.float32) + bb2_ref[...]
    z3 = zz.reshape(rb, K, 16)
    zmax = jnp.max(z3, axis=1, keepdims=True)
    e = jnp.exp(z3 - zmax)
    soft = e / jnp.sum(e, axis=1, keepdims=True)
    sa = jnp.concatenate([soft[:, :, :8]] * 16, axis=2)
    se = jnp.concatenate([soft[:, :, 8:]] * 16, axis=2)
    gv = lax.bitcast_convert_type(g_ref[...] & jnp.int32(-65536),
                                  jnp.float32).reshape(rb, K, 256)
    oa = jnp.sum(gv[:, :, :C] * sa, axis=1)
    oe = jnp.sum(gv[:, :, C:] * se, axis=1)
    out_ref[...] = jnp.concatenate([oa, oe], axis=1) + sp_ref[...]

  z = lambda i: (i, 0)
  c0 = lambda i: (0, 0)
  return pl.pallas_call(
      body,
      grid=(nb,),
      in_specs=[
          pl.BlockSpec((rb * K, 16), z),
          pl.BlockSpec((rb * K, 256), z),
          pl.BlockSpec((rb, 256), z),
          pl.BlockSpec((1, 16), c0), pl.BlockSpec((1, 16), c0),
          pl.BlockSpec((1, 16), c0), pl.BlockSpec((1, 16), c0),
          pl.BlockSpec((16, 16), c0), pl.BlockSpec((1, 16), c0),
      ],
      out_specs=pl.BlockSpec((rb, 256), z),
      out_shape=jax.ShapeDtypeStruct((N, 256), jnp.float32),
  )(y, g, sp, s3, s4, g2, b2, w2, bb2)


def kernel(coord, spa, spe, neighbor_indices, params):
  pa, pe = params['spa'], params['spe']
  spa_t = spa[0].T
  spe_t = spe[0].T
  coordp = jnp.pad(coord[0], ((0, 0), (0, 5)))
  coordw = jnp.pad(coord[0], ((0, 0), (0, C - 3)))
  idx = neighbor_indices[0].reshape(NK).astype(jnp.int32)

  cg = _sc_gather(coordw, idx)
  m1, m2 = _moments(cg)

  wqkv = jnp.stack([pa['wq'], pa['wk'], pa['wv'],
                    pe['wq'], pe['wk'], pe['wv']])
  bqkv = jnp.stack([pa['bq'], pa['bk'], pa['bv'],
                    pe['bq'], pe['bk'], pe['bv']])
  pad88 = lambda w: jnp.zeros((8, 8), jnp.float32).at[:3, :3].set(w)
  pad8 = lambda v: jnp.zeros((8,), jnp.float32).at[:3].set(v)
  wp1 = jnp.stack([pad88(pa['wp1']), pad88(pe['wp1'])])
  bp1 = jnp.stack([pad8(pa['bp1']), pad8(pe['bp1'])])
  gp = jnp.stack([pad8(pa['gp']), pad8(pe['gp'])])
  bpn = jnp.stack([pad8(pa['bpn']), pad8(pe['bpn'])])
  padw2 = lambda w: jnp.zeros((C, 8), jnp.float32).at[:, :3].set(w)
  wp2 = jnp.stack([padw2(pa['wp2']), padw2(pe['wp2'])])
  bp2 = jnp.stack([pa['bp2'], pe['bp2']])

  t, xq = _tables(spa_t, spe_t, coordp, m1, m2, wqkv, bqkv,
                  wp1, bp1, gp, bpn, wp2, bp2)
  g = _sc_gather(t, idx)
  s1, s2 = _stats(g, xq)

  g1 = jnp.concatenate([pa['gw1'], pe['gw1']]).reshape(1, 256)
  b1 = jnp.concatenate([pa['bw1'], pe['bw1']]).reshape(1, 256)
  w1 = (jnp.zeros((256, 16), jnp.float32)
        .at[:C, :8].set(pa['ww1'].T).at[C:, 8:].set(pe['ww1'].T))
  bb1 = jnp.concatenate([pa['bww1'], pe['bww1']]).reshape(1, 16)
  y, s3, s4 = _ypass(g, xq, s1, s2, g1, b1, w1, bb1)

  g2 = jnp.concatenate([pa['gw2'], pe['gw2']]).reshape(1, 16)
  b2 = jnp.concatenate([pa['bw2'], pe['bw2']]).reshape(1, 16)
  w2 = (jnp.zeros((16, 16), jnp.float32)
        .at[:8, :8].set(pa['ww2'].T).at[8:, 8:].set(pe['ww2'].T))
  bb2 = jnp.concatenate([pa['bww2'], pe['bww2']]).reshape(1, 16)
  sp = jnp.concatenate([spa_t, spe_t], axis=1)
  out = _final(y, g, sp, s3, s4, g2, b2, w2, bb2)
  return out.T[None]
